# baseline (speedup 1.0000x reference)
import functools

import jax
import jax.numpy as jnp
from jax import lax
from jax.experimental import pallas as pl
from jax.experimental.pallas import tpu as pltpu
from jax.experimental.pallas import tpu_sc as plsc

N = 1024
D = 64
B_TOTAL = 8 * 576
GRID = 8
BLK = B_TOTAL // GRID
EMPTY_VAL = 1.0 / D

NC = 2
NS = 16
NW = NC * NS
BPW = B_TOTAL // NW
CH = BPW // 2
LANES = 16


REFINE_TAU = 1e-4
REFINE_Q = 6


def _exact_ref_distance(vq_row, w):
    diff = vq_row - w
    d2 = diff * diff
    p = d2 + jnp.roll(d2, -4, axis=1)
    q = p + jnp.roll(p, -2, axis=1)
    r = q + jnp.roll(q, -1, axis=1)
    acc = r[:, 0:1]
    for g in range(1, D // 8):
        acc = acc + r[:, 8 * g:8 * g + 1]
    return acc


def _vq_tc_kernel(v_ref, w_ref, idx_ref, loss_ref, flag_ref):
    i = pl.program_id(0)
    vb = v_ref[...]
    w = w_ref[...]

    hp = jax.lax.Precision.HIGHEST
    wn_row = jax.lax.dot_general(
        jnp.ones((1, D), jnp.float32), w * w,
        (((1,), (1,)), ((), ())), precision=hp,
        preferred_element_type=jnp.float32)
    col0 = jax.lax.broadcasted_iota(jnp.int32, (1, N), 1)
    wn_row = jnp.where(col0 == 0, jnp.float32(3e38), wn_row)
    s = jax.lax.dot_general(
        vb * -2.0, w, (((1,), (1,)), ((), ())), precision=hp,
        preferred_element_type=jnp.float32)
    score = s + wn_row

    m = jnp.min(score, axis=1)
    colB = jax.lax.broadcasted_iota(jnp.int32, (BLK, N), 1)
    idx = jnp.min(jnp.where(score == m[:, None], colB, jnp.int32(N)),
                  axis=1).astype(jnp.int32)
    nonempty = jnp.any(vb != EMPTY_VAL, axis=1)
    idx_ref[...] = jnp.where(nonempty, idx, jnp.int32(0))[:, None]

    vn = jnp.sum(vb * vb, axis=1)
    part = jnp.sum(jnp.where(nonempty, m + vn, jnp.float32(0.0)))

    near = (score < (m + REFINE_TAU)[:, None]).astype(jnp.float32)
    nflag = jnp.sum(near, axis=1)
    flag = (nflag >= 2.0)
    flag_ref[...] = flag.astype(jnp.int32)[:, None]
    cnt = jnp.sum(flag.astype(jnp.int32))

    rows = jax.lax.broadcasted_iota(jnp.int32, (BLK, 1), 0)
    rown = jax.lax.broadcasted_iota(jnp.int32, (N, 1), 0)

    def _refine_one(_, carry):
        f = flag_ref[...]
        r = jnp.min(jnp.where(f > 0, rows, jnp.int32(BLK - 1)))
        flag_ref[pl.ds(r, 1), :] = jnp.zeros((1, 1), jnp.int32)
        vq = v_ref[pl.ds(r, 1), :]
        dist = _exact_ref_distance(vq, w)
        dist = jnp.where(rown == 0, jnp.float32(3e38), dist)
        md = jnp.min(dist)
        jq = jnp.min(jnp.where(dist == md, rown, jnp.int32(N)))
        ne_r = jnp.any(vq != EMPTY_VAL)
        newidx = jnp.where(ne_r, jq, jnp.int32(0))
        idx_ref[pl.ds(r, 1), :] = newidx[None, None]
        return carry

    jax.lax.fori_loop(0, cnt, _refine_one, jnp.int32(0))

    @pl.when(i == 0)
    def _():
        loss_ref[...] = jnp.zeros((1, 1), jnp.float32)

    loss_ref[...] = loss_ref[...] + part

    @pl.when(i == GRID - 1)
    def _():
        loss_ref[...] = loss_ref[...] / jnp.float32(B_TOTAL * D)


def _sc_gather_body(table_hbm, idx_hbm, out_hbm,
                    idx_v0, idx_v1, rows_v, sem0, sem1):
    wid = lax.axis_index("s") * NC + lax.axis_index("c")
    base = wid * BPW
    pltpu.sync_copy(idx_hbm.at[pl.ds(base, CH)], idx_v0)
    pltpu.sync_copy(idx_hbm.at[pl.ds(base + CH, CH)], idx_v1)
    cp0 = pltpu.async_copy(table_hbm.at[idx_v0], rows_v.at[pl.ds(0, CH)], sem0)
    cp1 = pltpu.async_copy(table_hbm.at[idx_v1], rows_v.at[pl.ds(CH, CH)],
                           sem1)
    cp0.wait()
    cp1.wait()
    pltpu.sync_copy(rows_v, out_hbm.at[pl.ds(base, BPW)])


@functools.cache
def _get_sc_gather():
    return pl.kernel(
        _sc_gather_body,
        out_type=jax.ShapeDtypeStruct((B_TOTAL, D), jnp.float32),
        mesh=plsc.VectorSubcoreMesh(core_axis_name="c", subcore_axis_name="s",
                                    num_cores=NC, num_subcores=NS),
        compiler_params=pltpu.CompilerParams(use_tc_tiling_on_sc=False),
        scratch_types=[
            pltpu.VMEM((CH,), jnp.int32),
            pltpu.VMEM((CH,), jnp.int32),
            pltpu.VMEM((BPW, D), jnp.float32),
            pltpu.SemaphoreType.DMA,
            pltpu.SemaphoreType.DMA,
        ],
    )


@jax.jit
def _vq(vf, vectors):
    idx, loss = pl.pallas_call(
        _vq_tc_kernel,
        grid=(GRID,),
        in_specs=[
            pl.BlockSpec((BLK, D), lambda i: (i, 0)),
            pl.BlockSpec((N, D), lambda i: (0, 0)),
        ],
        out_specs=[
            pl.BlockSpec((BLK, 1), lambda i: (i, 0)),
            pl.BlockSpec((1, 1), lambda i: (0, 0)),
        ],
        out_shape=[
            jax.ShapeDtypeStruct((B_TOTAL, 1), jnp.int32),
            jax.ShapeDtypeStruct((1, 1), jnp.float32),
        ],
        scratch_shapes=[pltpu.VMEM((BLK, 1), jnp.int32)],
        compiler_params=pltpu.CompilerParams(
            dimension_semantics=("arbitrary",)),
    )(vf, vectors)
    idx_flat = idx.reshape(B_TOTAL)
    out = _get_sc_gather()(vectors, idx_flat)
    return out, idx_flat, loss[0, 0]


def kernel(v, vectors):
    lead = v.shape[:-1]
    vf = v.reshape(-1, D)
    out, idx, loss = _vq(vf, vectors)
    used = jnp.array([0], dtype=jnp.int32)
    return (out.reshape(*lead, D), idx.reshape(lead), loss, used)


# speedup vs baseline: 1.0515x; 1.0515x over previous
import functools

import jax
import jax.numpy as jnp
from jax import lax
from jax.experimental import pallas as pl
from jax.experimental.pallas import tpu as pltpu
from jax.experimental.pallas import tpu_sc as plsc

N = 1024
D = 64
B_TOTAL = 8 * 576
GRID = 2
BLK = B_TOTAL // GRID
EMPTY_VAL = 1.0 / D

NC = 2
NS = 16
NW = NC * NS
BPW = B_TOTAL // NW
CH = BPW // 2
LANES = 16


REFINE_TAU = 1e-4
REFINE_Q = 6


def _exact_ref_distance(vq_row, w):
    diff = vq_row - w
    d2 = diff * diff
    p = d2 + jnp.roll(d2, -4, axis=1)
    q = p + jnp.roll(p, -2, axis=1)
    r = q + jnp.roll(q, -1, axis=1)
    acc = r[:, 0:1]
    for g in range(1, D // 8):
        acc = acc + r[:, 8 * g:8 * g + 1]
    return acc


def _vq_tc_kernel(v_ref, w_ref, idx_ref, loss_ref, flag_ref):
    i = pl.program_id(0)
    vb = v_ref[...]
    w = w_ref[...]

    hp = jax.lax.Precision.HIGHEST
    wn_row = jax.lax.dot_general(
        jnp.ones((1, D), jnp.float32), w * w,
        (((1,), (1,)), ((), ())), precision=hp,
        preferred_element_type=jnp.float32)
    col0 = jax.lax.broadcasted_iota(jnp.int32, (1, N), 1)
    wn_row = jnp.where(col0 == 0, jnp.float32(3e38), wn_row)
    s = jax.lax.dot_general(
        vb * -2.0, w, (((1,), (1,)), ((), ())), precision=hp,
        preferred_element_type=jnp.float32)
    score = s + wn_row

    m = jnp.min(score, axis=1)
    colB = jax.lax.broadcasted_iota(jnp.int32, (BLK, N), 1)
    idx = jnp.min(jnp.where(score == m[:, None], colB, jnp.int32(N)),
                  axis=1).astype(jnp.int32)
    nonempty = jnp.any(vb != EMPTY_VAL, axis=1)
    idx_ref[...] = jnp.where(nonempty, idx, jnp.int32(0))[:, None]

    vn = jnp.sum(vb * vb, axis=1)
    part = jnp.sum(jnp.where(nonempty, m + vn, jnp.float32(0.0)))

    near = (score < (m + REFINE_TAU)[:, None]).astype(jnp.float32)
    nflag = jnp.sum(near, axis=1)
    flag = (nflag >= 2.0)
    flag_ref[...] = flag.astype(jnp.int32)[:, None]
    cnt = jnp.sum(flag.astype(jnp.int32))

    rows = jax.lax.broadcasted_iota(jnp.int32, (BLK, 1), 0)
    rown = jax.lax.broadcasted_iota(jnp.int32, (N, 1), 0)

    def _refine_one(_, carry):
        f = flag_ref[...]
        r = jnp.min(jnp.where(f > 0, rows, jnp.int32(BLK - 1)))
        flag_ref[pl.ds(r, 1), :] = jnp.zeros((1, 1), jnp.int32)
        vq = v_ref[pl.ds(r, 1), :]
        dist = _exact_ref_distance(vq, w)
        dist = jnp.where(rown == 0, jnp.float32(3e38), dist)
        md = jnp.min(dist)
        jq = jnp.min(jnp.where(dist == md, rown, jnp.int32(N)))
        ne_r = jnp.any(vq != EMPTY_VAL)
        newidx = jnp.where(ne_r, jq, jnp.int32(0))
        idx_ref[pl.ds(r, 1), :] = newidx[None, None]
        return carry

    jax.lax.fori_loop(0, cnt, _refine_one, jnp.int32(0))

    @pl.when(i == 0)
    def _():
        loss_ref[...] = jnp.zeros((1, 1), jnp.float32)

    loss_ref[...] = loss_ref[...] + part

    @pl.when(i == GRID - 1)
    def _():
        loss_ref[...] = loss_ref[...] / jnp.float32(B_TOTAL * D)


def _sc_gather_body(table_hbm, idx_hbm, out_hbm,
                    idx_v0, idx_v1, rows_v, sem0, sem1):
    wid = lax.axis_index("s") * NC + lax.axis_index("c")
    base = wid * BPW
    pltpu.sync_copy(idx_hbm.at[pl.ds(base, CH)], idx_v0)
    pltpu.sync_copy(idx_hbm.at[pl.ds(base + CH, CH)], idx_v1)
    cp0 = pltpu.async_copy(table_hbm.at[idx_v0], rows_v.at[pl.ds(0, CH)], sem0)
    cp1 = pltpu.async_copy(table_hbm.at[idx_v1], rows_v.at[pl.ds(CH, CH)],
                           sem1)
    cp0.wait()
    cp1.wait()
    pltpu.sync_copy(rows_v, out_hbm.at[pl.ds(base, BPW)])


@functools.cache
def _get_sc_gather():
    return pl.kernel(
        _sc_gather_body,
        out_type=jax.ShapeDtypeStruct((B_TOTAL, D), jnp.float32),
        mesh=plsc.VectorSubcoreMesh(core_axis_name="c", subcore_axis_name="s",
                                    num_cores=NC, num_subcores=NS),
        compiler_params=pltpu.CompilerParams(use_tc_tiling_on_sc=False),
        scratch_types=[
            pltpu.VMEM((CH,), jnp.int32),
            pltpu.VMEM((CH,), jnp.int32),
            pltpu.VMEM((BPW, D), jnp.float32),
            pltpu.SemaphoreType.DMA,
            pltpu.SemaphoreType.DMA,
        ],
    )


@jax.jit
def _vq(vf, vectors):
    idx, loss = pl.pallas_call(
        _vq_tc_kernel,
        grid=(GRID,),
        in_specs=[
            pl.BlockSpec((BLK, D), lambda i: (i, 0)),
            pl.BlockSpec((N, D), lambda i: (0, 0)),
        ],
        out_specs=[
            pl.BlockSpec((BLK, 1), lambda i: (i, 0)),
            pl.BlockSpec((1, 1), lambda i: (0, 0)),
        ],
        out_shape=[
            jax.ShapeDtypeStruct((B_TOTAL, 1), jnp.int32),
            jax.ShapeDtypeStruct((1, 1), jnp.float32),
        ],
        scratch_shapes=[pltpu.VMEM((BLK, 1), jnp.int32)],
        compiler_params=pltpu.CompilerParams(
            dimension_semantics=("arbitrary",)),
    )(vf, vectors)
    idx_flat = idx.reshape(B_TOTAL)
    out = _get_sc_gather()(vectors, idx_flat)
    return out, idx_flat, loss[0, 0]


def kernel(v, vectors):
    lead = v.shape[:-1]
    vf = v.reshape(-1, D)
    out, idx, loss = _vq(vf, vectors)
    used = jnp.array([0], dtype=jnp.int32)
    return (out.reshape(*lead, D), idx.reshape(lead), loss, used)
